# baseline (speedup 1.0000x reference)
import functools

import jax
import jax.numpy as jnp
from jax import lax
from jax.experimental import pallas as pl
from jax.experimental.pallas import tpu as pltpu
from jax.experimental.pallas import tpu_sc as plsc

N_NODES = 10000
N_EDGES = 320000
D = 128
DH = D // 2

NC = 2
NS = 16
NW = NC * NS
K = 80
EPTA = N_EDGES // NS
NCA = EPTA // K
EPTD = N_EDGES // NW
NCD = EPTD // K
NB = 8
NP = 4
RPT = N_NODES // NS
ZRD = 125

_mesh = plsc.VectorSubcoreMesh(
    core_axis_name="c", subcore_axis_name="s", num_cores=NC, num_subcores=NS
)
_sc_params = pltpu.CompilerParams(use_tc_tiling_on_sc=False)


def _zero_fill(ref, rows, width):
    zero16 = jnp.zeros((16,), jnp.float32)

    @pl.loop(0, rows)
    def _(i):
        for l in range(width // 16):
            ref[i, pl.ds(l * 16, 16)] = zero16


@functools.partial(
    pl.kernel,
    out_type=jax.ShapeDtypeStruct((NC, N_NODES, 16), jnp.float32),
    mesh=_mesh,
    compiler_params=_sc_params,
    scratch_types=[
        pltpu.VMEM((NCD, K), jnp.int32),
        pltpu.VMEM((K, 16), jnp.float32),
        pltpu.VMEM((ZRD, 16), jnp.float32),
        pltpu.VMEM_SHARED((N_NODES, 16), jnp.float32),
        pltpu.SemaphoreType.DMA,
    ],
)
def _sc_degree(dst_hbm, out_hbm, dstb, ones_v, zero_v, acc_s, sem):
    cid = lax.axis_index("c")
    sid = lax.axis_index("s")

    one16 = jnp.ones((16,), jnp.float32)

    @pl.loop(0, K)
    def _(i):
        ones_v[i, :] = one16

    _zero_fill(zero_v, ZRD, 16)

    pltpu.sync_copy(dst_hbm.at[sid, pl.ds(cid * NCD, NCD)], dstb)

    @pl.loop(0, RPT // ZRD)
    def _(i):
        pltpu.sync_copy(zero_v, acc_s.at[pl.ds(sid * RPT + i * ZRD, ZRD)])

    plsc.subcore_barrier()

    @pl.loop(0, NCD // 25)
    def _(b):
        @pl.loop(0, 25)
        def _(j):
            pltpu.async_copy(ones_v, acc_s.at[dstb.at[b * 25 + j]], sem,
                             add=True)

        @pl.loop(0, 25)
        def _(j):
            pltpu.make_async_copy(ones_v, acc_s.at[dstb.at[0]], sem).wait()

    plsc.subcore_barrier()
    pltpu.sync_copy(
        acc_s.at[pl.ds(sid * RPT, RPT)], out_hbm.at[cid, pl.ds(sid * RPT, RPT)]
    )


@functools.partial(
    pl.kernel,
    out_type=jax.ShapeDtypeStruct((NC, N_NODES, DH), jnp.float32),
    mesh=_mesh,
    compiler_params=_sc_params,
    scratch_types=[
        pltpu.VMEM((NCA, K), jnp.int32),
        pltpu.VMEM((NCA, K), jnp.int32),
        pltpu.VMEM((NB, K, DH), jnp.float32),
        pltpu.VMEM_SHARED((N_NODES, DH), jnp.float32),
        [pltpu.SemaphoreType.DMA] * NB,
        [pltpu.SemaphoreType.DMA] * NB,
    ],
)
def _sc_aggregate(g_hbm, src_hbm, dst_hbm, out_hbm,
                  srcb, dstb, rows_v, acc_s, gsems, ssems):
    cid = lax.axis_index("c")
    sid = lax.axis_index("s")

    pltpu.sync_copy(src_hbm.at[cid, sid], srcb)
    pltpu.sync_copy(dst_hbm.at[sid], dstb)

    _zero_fill(rows_v.at[0], K, DH)

    @pl.loop(0, 7)
    def _(i):
        pltpu.sync_copy(rows_v.at[0], acc_s.at[pl.ds(sid * RPT + i * K, K)])

    pltpu.sync_copy(
        rows_v.at[0, pl.ds(0, RPT - 7 * K)],
        acc_s.at[pl.ds(sid * RPT + 7 * K, RPT - 7 * K)],
    )

    plsc.subcore_barrier()

    def _start_gather(j, s):
        pltpu.async_copy(g_hbm.at[srcb.at[j]], rows_v.at[s], gsems[s])

    def _wait_gather(j, s):
        pltpu.make_async_copy(
            g_hbm.at[srcb.at[j]], rows_v.at[s], gsems[s]
        ).wait()

    def _start_scatter(j, s):
        pltpu.async_copy(rows_v.at[s], acc_s.at[dstb.at[j]], ssems[s],
                         add=True)

    def _wait_scatter(j, s):
        pltpu.make_async_copy(
            rows_v.at[s], acc_s.at[dstb.at[j]], ssems[s]
        ).wait()

    for s in range(NP):
        _start_gather(s, s)

    def _step(j, s):
        _wait_gather(j, s)
        _start_scatter(j, s)
        jn = j + NP

        @pl.when(jn < NCA)
        def _():
            sn = (s + NP) % NB

            @pl.when(jn >= NB)
            def _():
                _wait_scatter(jn - NB, sn)

            _start_gather(jn, sn)

    NGRP = NCA // NB
    NTAIL = NCA - NGRP * NB

    @pl.loop(0, NGRP)
    def _(g):
        for s in range(NB):
            _step(g * NB + s, s)

    for s in range(NTAIL):
        _step(NGRP * NB + s, s)

    for m in range(NCA - NB, NCA):
        _wait_scatter(m, m % NB)

    plsc.subcore_barrier()
    pltpu.sync_copy(
        acc_s.at[pl.ds(sid * RPT, RPT)], out_hbm.at[cid, pl.ds(sid * RPT, RPT)]
    )


_BLK = 1000
_GRID = N_NODES // _BLK


def _dis_block(degp):
    deg = degp[0] + degp[1] + 1.0
    return lax.rsqrt(deg)[:, 0:1]


def _split_store(out_ref, v):
    out_ref[0] = v[:, :DH]
    out_ref[1] = v[:, DH:]


def _joined(pair_ref):
    return jnp.concatenate([pair_ref[0], pair_ref[1]], axis=-1)


def _tc_stage1(x_ref, w_ref, degp_ref, g_ref):
    h = jnp.dot(x_ref[...], w_ref[...], preferred_element_type=jnp.float32)
    _split_store(g_ref, h * _dis_block(degp_ref[...]))


def _tc_stage2(acc_ref, g_ref, degp_ref, b_ref, w_ref, out_ref):
    dis = _dis_block(degp_ref[...])
    agg = _joined(acc_ref) + _joined(g_ref)
    h = jnp.maximum(agg * dis + b_ref[...], 0.0)
    g2 = jnp.dot(h, w_ref[...], preferred_element_type=jnp.float32) * dis
    _split_store(out_ref, g2)


def _tc_stage3(acc_ref, g_ref, degp_ref, b_ref, out_ref):
    dis = _dis_block(degp_ref[...])
    agg = _joined(acc_ref) + _joined(g_ref)
    out_ref[...] = agg * dis + b_ref[...]


def _row_spec(width):
    return pl.BlockSpec((_BLK, width), lambda i: (i, 0))


def _full_spec(shape):
    return pl.BlockSpec(shape, lambda i: tuple(0 for _ in shape))


def _pair_spec(width):
    return pl.BlockSpec((NC, _BLK, width), lambda i: (0, i, 0))


def kernel(x, edge_index, W1, b1, W2, b2):
    src = edge_index[0].astype(jnp.int32).reshape(NS, NCA, K)
    dst = edge_index[1].astype(jnp.int32).reshape(NS, NCA, K)
    srcx = jnp.stack([src, src + N_NODES])
    b1r = b1.reshape(1, D)
    b2r = b2.reshape(1, D)

    degp = _sc_degree(dst)

    g1 = pl.pallas_call(
        _tc_stage1,
        grid=(_GRID,),
        in_specs=[_row_spec(D), _full_spec((D, D)), _pair_spec(16)],
        out_specs=_pair_spec(DH),
        out_shape=jax.ShapeDtypeStruct((NC, N_NODES, DH), jnp.float32),
    )(x, W1, degp)

    acc1 = _sc_aggregate(g1.reshape(NC * N_NODES, DH), srcx, dst)

    g2 = pl.pallas_call(
        _tc_stage2,
        grid=(_GRID,),
        in_specs=[
            _pair_spec(DH),
            _pair_spec(DH),
            _pair_spec(16),
            _full_spec((1, D)),
            _full_spec((D, D)),
        ],
        out_specs=_pair_spec(DH),
        out_shape=jax.ShapeDtypeStruct((NC, N_NODES, DH), jnp.float32),
    )(acc1, g1, degp, b1r, W2)

    acc2 = _sc_aggregate(g2.reshape(NC * N_NODES, DH), srcx, dst)

    out = pl.pallas_call(
        _tc_stage3,
        grid=(_GRID,),
        in_specs=[
            _pair_spec(DH),
            _pair_spec(DH),
            _pair_spec(16),
            _full_spec((1, D)),
        ],
        out_specs=_row_spec(D),
        out_shape=jax.ShapeDtypeStruct((N_NODES, D), jnp.float32),
    )(acc2, g2, degp, b2r)

    return out


# speedup vs baseline: 1.1842x; 1.1842x over previous
import functools

import jax
import jax.numpy as jnp
from jax import lax
from jax.experimental import pallas as pl
from jax.experimental.pallas import tpu as pltpu
from jax.experimental.pallas import tpu_sc as plsc

N_NODES = 10000
N_EDGES = 320000
D = 128

NC = 2
NS = 16
NW = NC * NS
K = 80
EPT = N_EDGES // NW
NCH = EPT // K
NB = 3
NP = 2
RPT = N_NODES // NS
ZRD = 125

_mesh = plsc.VectorSubcoreMesh(
    core_axis_name="c", subcore_axis_name="s", num_cores=NC, num_subcores=NS
)
_sc_params = pltpu.CompilerParams(use_tc_tiling_on_sc=False)


def _zero_fill(ref, rows, width):
    zero16 = jnp.zeros((16,), jnp.float32)

    @pl.loop(0, rows)
    def _(i):
        for l in range(width // 16):
            ref[i, pl.ds(l * 16, 16)] = zero16


@functools.partial(
    pl.kernel,
    out_type=jax.ShapeDtypeStruct((NC, N_NODES, 16), jnp.float32),
    mesh=_mesh,
    compiler_params=_sc_params,
    scratch_types=[
        pltpu.VMEM((NCH, K), jnp.int32),
        pltpu.VMEM((K, 16), jnp.float32),
        pltpu.VMEM((ZRD, 16), jnp.float32),
        pltpu.VMEM_SHARED((N_NODES, 16), jnp.float32),
        pltpu.SemaphoreType.DMA,
    ],
)
def _sc_degree(dst_hbm, out_hbm, dstb, ones_v, zero_v, acc_s, sem):
    cid = lax.axis_index("c")
    sid = lax.axis_index("s")
    wid = cid * NS + sid

    one16 = jnp.ones((16,), jnp.float32)

    @pl.loop(0, K)
    def _(i):
        ones_v[i, :] = one16

    _zero_fill(zero_v, ZRD, 16)

    pltpu.sync_copy(dst_hbm.at[wid], dstb)

    @pl.loop(0, RPT // ZRD)
    def _(i):
        pltpu.sync_copy(zero_v, acc_s.at[pl.ds(sid * RPT + i * ZRD, ZRD)])

    plsc.subcore_barrier()

    @pl.loop(0, NCH // 25)
    def _(b):
        @pl.loop(0, 25)
        def _(j):
            pltpu.async_copy(ones_v, acc_s.at[dstb.at[b * 25 + j]], sem,
                             add=True)

        @pl.loop(0, 25)
        def _(j):
            pltpu.make_async_copy(ones_v, acc_s.at[dstb.at[0]], sem).wait()

    plsc.subcore_barrier()
    pltpu.sync_copy(
        acc_s.at[pl.ds(sid * RPT, RPT)], out_hbm.at[cid, pl.ds(sid * RPT, RPT)]
    )


@functools.partial(
    pl.kernel,
    out_type=jax.ShapeDtypeStruct((NC, N_NODES, D), jnp.float32),
    mesh=_mesh,
    compiler_params=_sc_params,
    scratch_types=[
        pltpu.VMEM((NCH, K), jnp.int32),
        pltpu.VMEM((NCH, K), jnp.int32),
        pltpu.VMEM((NB, K, D), jnp.float32),
        pltpu.VMEM_SHARED((N_NODES, D), jnp.float32),
        [pltpu.SemaphoreType.DMA] * NB,
        [pltpu.SemaphoreType.DMA] * NB,
    ],
)
def _sc_aggregate(g_hbm, src_hbm, dst_hbm, out_hbm,
                  srcb, dstb, rows_v, acc_s, gsems, ssems):
    cid = lax.axis_index("c")
    sid = lax.axis_index("s")
    wid = cid * NS + sid

    pltpu.sync_copy(src_hbm.at[wid], srcb)
    pltpu.sync_copy(dst_hbm.at[wid], dstb)

    _zero_fill(rows_v.at[0], K, D)

    @pl.loop(0, 7)
    def _(i):
        pltpu.sync_copy(rows_v.at[0], acc_s.at[pl.ds(sid * RPT + i * K, K)])

    pltpu.sync_copy(
        rows_v.at[0, pl.ds(0, RPT - 7 * K)],
        acc_s.at[pl.ds(sid * RPT + 7 * K, RPT - 7 * K)],
    )

    plsc.subcore_barrier()

    def _start_gather(j, s):
        pltpu.async_copy(g_hbm.at[srcb.at[j]], rows_v.at[s], gsems[s])

    def _wait_gather(j, s):
        pltpu.make_async_copy(
            g_hbm.at[srcb.at[j]], rows_v.at[s], gsems[s]
        ).wait()

    def _start_scatter(j, s):
        pltpu.async_copy(rows_v.at[s], acc_s.at[dstb.at[j]], ssems[s],
                         add=True)

    def _wait_scatter(j, s):
        pltpu.make_async_copy(
            rows_v.at[s], acc_s.at[dstb.at[j]], ssems[s]
        ).wait()

    for s in range(NP):
        _start_gather(s, s)

    def _step(j, s):
        _wait_gather(j, s)
        _start_scatter(j, s)
        jn = j + NP

        @pl.when(jn < NCH)
        def _():
            sn = (s + NP) % NB

            @pl.when(jn >= NB)
            def _():
                _wait_scatter(jn - NB, sn)

            _start_gather(jn, sn)

    NGRP = NCH // NB
    NTAIL = NCH - NGRP * NB

    @pl.loop(0, NGRP)
    def _(g):
        for s in range(NB):
            _step(g * NB + s, s)

    for s in range(NTAIL):
        _step(NGRP * NB + s, s)

    for m in range(NCH - NB, NCH):
        _wait_scatter(m, m % NB)

    plsc.subcore_barrier()
    pltpu.sync_copy(
        acc_s.at[pl.ds(sid * RPT, RPT)], out_hbm.at[cid, pl.ds(sid * RPT, RPT)]
    )


_BLK = 1000
_GRID = N_NODES // _BLK


def _dis_block(degp):
    deg = degp[0] + degp[1] + 1.0
    return lax.rsqrt(deg)[:, 0:1]


def _tc_stage1(x_ref, w_ref, degp_ref, g_ref):
    h = jnp.dot(x_ref[...], w_ref[...], preferred_element_type=jnp.float32)
    g_ref[...] = h * _dis_block(degp_ref[...])


def _tc_stage2(acc_ref, g_ref, degp_ref, b_ref, w_ref, out_ref):
    dis = _dis_block(degp_ref[...])
    agg = acc_ref[0] + acc_ref[1] + g_ref[...]
    h = jnp.maximum(agg * dis + b_ref[...], 0.0)
    out_ref[...] = (
        jnp.dot(h, w_ref[...], preferred_element_type=jnp.float32) * dis
    )


def _tc_stage3(acc_ref, g_ref, degp_ref, b_ref, out_ref):
    dis = _dis_block(degp_ref[...])
    agg = acc_ref[0] + acc_ref[1] + g_ref[...]
    out_ref[...] = agg * dis + b_ref[...]


def _row_spec(width):
    return pl.BlockSpec((_BLK, width), lambda i: (i, 0))


def _full_spec(shape):
    return pl.BlockSpec(shape, lambda i: tuple(0 for _ in shape))


def _pair_spec(width):
    return pl.BlockSpec((NC, _BLK, width), lambda i: (0, i, 0))


def kernel(x, edge_index, W1, b1, W2, b2):
    src = edge_index[0].astype(jnp.int32).reshape(NW, NCH, K)
    dst = edge_index[1].astype(jnp.int32).reshape(NW, NCH, K)
    b1r = b1.reshape(1, D)
    b2r = b2.reshape(1, D)

    degp = _sc_degree(dst)

    g1 = pl.pallas_call(
        _tc_stage1,
        grid=(_GRID,),
        in_specs=[_row_spec(D), _full_spec((D, D)), _pair_spec(16)],
        out_specs=_row_spec(D),
        out_shape=jax.ShapeDtypeStruct((N_NODES, D), jnp.float32),
    )(x, W1, degp)

    acc1 = _sc_aggregate(g1, src, dst)

    g2 = pl.pallas_call(
        _tc_stage2,
        grid=(_GRID,),
        in_specs=[
            _pair_spec(D),
            _row_spec(D),
            _pair_spec(16),
            _full_spec((1, D)),
            _full_spec((D, D)),
        ],
        out_specs=_row_spec(D),
        out_shape=jax.ShapeDtypeStruct((N_NODES, D), jnp.float32),
    )(acc1, g1, degp, b1r, W2)

    acc2 = _sc_aggregate(g2, src, dst)

    out = pl.pallas_call(
        _tc_stage3,
        grid=(_GRID,),
        in_specs=[
            _pair_spec(D),
            _row_spec(D),
            _pair_spec(16),
            _full_spec((1, D)),
        ],
        out_specs=_row_spec(D),
        out_shape=jax.ShapeDtypeStruct((N_NODES, D), jnp.float32),
    )(acc2, g2, degp, b2r)

    return out
